# Initial kernel scaffold; baseline (speedup 1.0000x reference)
import jax
import jax.numpy as jnp
from jax.experimental import pallas as pl


def kernel(user_table, business_table, city_table, category_table, Wq, Wk, Wv, user_idx, business_neigh_idx, city_idx, category_idx):
    raise NotImplementedError("write your pallas kernel here")


# speedup vs baseline: 11.6248x; 11.6248x over previous
import functools

import jax
import jax.numpy as jnp
from jax import lax
from jax.experimental import pallas as pl
from jax.experimental.pallas import tpu as pltpu
from jax.experimental.pallas import tpu_sc as plsc

B = 16384
H = 50
D = 64

NC = 2
NS = 16
NW = NC * NS
L = 16

RPW = B // NW
NBUF = 4
RPC = 2
CPW = RPW // RPC
IDX_COLS = RPC * H

_mesh = plsc.VectorSubcoreMesh(core_axis_name="c", subcore_axis_name="s")
_SC_PARAMS = pltpu.CompilerParams(use_tc_tiling_on_sc=False,
                                  needs_layout_passes=False)


def _wid():
    return lax.axis_index("s") * NC + lax.axis_index("c")


_GDN = lax.GatherDimensionNumbers(
    offset_dims=(), collapsed_slice_dims=(0,), start_index_map=(0,))


def _bcast_sum(v):
    cs = plsc.cumsum(v)
    idx = jnp.full((L,), L - 1, jnp.int32)
    return lax.gather(cs, idx[:, None], dimension_numbers=_GDN,
                      slice_sizes=(1,),
                      mode=lax.GatherScatterMode.PROMISE_IN_BOUNDS)


@functools.partial(
    pl.kernel,
    out_type=(
        jax.ShapeDtypeStruct((B, D), jnp.float32),
        jax.ShapeDtypeStruct((B, D), jnp.float32),
    ),
    mesh=_mesh,
    scratch_types=[
        pltpu.VMEM((4, 128), jnp.int32),
        pltpu.VMEM((RPW, D), jnp.float32),
        pltpu.VMEM((RPW, D), jnp.float32),
        pltpu.SemaphoreType.DMA((4,)),
    ],
    compiler_params=_SC_PARAMS,
)
def _ucg_kernel(utab, ctab, gtab, uidx, cidx, gidx, u_out, s_out,
                ib, rows, tmp, sems):
    wid = _wid()

    def gather512(tab, idx2, dst):
        pltpu.sync_copy(idx2.at[pl.ds(wid * 4, 4)], ib)
        for j in range(4):
            pltpu.async_copy(tab.at[ib.at[j]], dst.at[pl.ds(128 * j, 128)],
                             sems.at[j])
        for j in range(4):
            pltpu.make_async_copy(tab.at[ib.at[j]],
                                  dst.at[pl.ds(128 * j, 128)],
                                  sems.at[j]).wait()

    def accumulate():
        def body(r, _):
            for j in range(4):
                sl = pl.ds(L * j, L)
                rows[r, sl] = rows[r, sl] + tmp[r, sl]
            return ()
        lax.fori_loop(0, RPW, body, ())

    base = wid * RPW
    gather512(utab, uidx, rows)
    pltpu.sync_copy(rows, u_out.at[pl.ds(base, RPW)])
    gather512(ctab, cidx, tmp)
    accumulate()
    gather512(gtab, gidx, tmp)
    accumulate()
    pltpu.sync_copy(rows, s_out.at[pl.ds(base, RPW)])


def _proj_body(u_ref, wq_ref, wk_ref, o_ref):
    m = lax.dot_general(wq_ref[...], wk_ref[...], (((1,), (1,)), ((), ())),
                        preferred_element_type=jnp.float32)
    o_ref[...] = lax.dot_general(u_ref[...], m, (((1,), (0,)), ((), ())),
                                 preferred_element_type=jnp.float32) * 0.125


_proj = pl.pallas_call(
    _proj_body, out_shape=jax.ShapeDtypeStruct((B, D), jnp.float32))


@functools.partial(
    pl.kernel,
    out_type=jax.ShapeDtypeStruct((B, D), jnp.float32),
    mesh=_mesh,
    scratch_types=[
        pltpu.VMEM((CPW, IDX_COLS), jnp.int32),
        pltpu.VMEM((RPW, D), jnp.float32),
        pltpu.VMEM((NBUF, IDX_COLS, D), jnp.float32),
        pltpu.VMEM((RPW, D), jnp.float32),
        pltpu.SemaphoreType.DMA((NBUF,)),
    ],
    compiler_params=_SC_PARAMS,
)
def _attn_kernel(btab, bidx, p_in, out_hbm, idxv, pbuf, gbuf, obuf, sems):
    wid = _wid()
    pltpu.sync_copy(bidx.at[pl.ds(wid * CPW, CPW)], idxv)
    pltpu.sync_copy(p_in.at[pl.ds(wid * RPW, RPW)], pbuf)

    def start(c, k):
        pltpu.async_copy(btab.at[idxv.at[c]], gbuf.at[k], sems.at[k])

    def wait(c, k):
        pltpu.make_async_copy(btab.at[idxv.at[c]], gbuf.at[k],
                              sems.at[k]).wait()

    def compute(c, k):
        for r in range(RPC):
            row = RPC * c + r
            pv = tuple(pbuf[row, pl.ds(L * j, L)] for j in range(4))
            zero = jnp.zeros((L,), jnp.float32)

            def hbody(it, carry):
                a0, a1, a2, a3, den = carry
                for t in range(10):
                    hrow = r * H + it * 10 + t
                    b0 = gbuf[k, hrow, pl.ds(0, L)]
                    b1 = gbuf[k, hrow, pl.ds(L, L)]
                    b2 = gbuf[k, hrow, pl.ds(2 * L, L)]
                    b3 = gbuf[k, hrow, pl.ds(3 * L, L)]
                    dv = (pv[0] * b0 + pv[1] * b1) + (pv[2] * b2 + pv[3] * b3)
                    e = jnp.exp(_bcast_sum(dv))
                    den = den + e
                    a0 = a0 + e * b0
                    a1 = a1 + e * b1
                    a2 = a2 + e * b2
                    a3 = a3 + e * b3
                return a0, a1, a2, a3, den

            a0, a1, a2, a3, den = lax.fori_loop(
                0, H // 10, hbody, (zero, zero, zero, zero, zero))
            r_den = 1.0 / den
            obuf[row, pl.ds(0, L)] = a0 * r_den
            obuf[row, pl.ds(L, L)] = a1 * r_den
            obuf[row, pl.ds(2 * L, L)] = a2 * r_den
            obuf[row, pl.ds(3 * L, L)] = a3 * r_den

    for k in range(NBUF):
        start(k, k)

    def loop_body(i, _):
        for k in range(NBUF):
            c = i * NBUF + k
            wait(c, k)
            compute(c, k)

            @pl.when(i < CPW // NBUF - 1)
            def _():
                start(c + NBUF, k)
        return ()

    lax.fori_loop(0, CPW // NBUF, loop_body, ())
    pltpu.sync_copy(obuf, out_hbm.at[pl.ds(wid * RPW, RPW)])


def _final_body(s_ref, a_ref, wv_ref, o_ref):
    o_ref[...] = s_ref[...] + lax.dot_general(
        a_ref[...], wv_ref[...], (((1,), (0,)), ((), ())),
        preferred_element_type=jnp.float32)


_final = pl.pallas_call(
    _final_body, out_shape=jax.ShapeDtypeStruct((B, D), jnp.float32))


def kernel(user_table, business_table, city_table, category_table,
           Wq, Wk, Wv, user_idx, business_neigh_idx, city_idx, category_idx):
    uidx = user_idx.astype(jnp.int32).reshape(B // 128, 128)
    cidx = city_idx.astype(jnp.int32).reshape(B // 128, 128)
    gidx = category_idx.astype(jnp.int32).reshape(B // 128, 128)
    bidx = business_neigh_idx.astype(jnp.int32).reshape(
        B // RPC, IDX_COLS)

    u, s = _ucg_kernel(user_table, city_table, category_table,
                       uidx, cidx, gidx)
    p = _proj(u, Wq, Wk)
    anorm = _attn_kernel(business_table, bidx, p)
    return _final(s, anorm, Wv)
